# baseline (speedup 1.0000x reference)
import jax
import jax.numpy as jnp
from jax.experimental import pallas as pl
from jax.experimental.pallas import tpu as pltpu
from jax.experimental.pallas import tpu_sc as plsc

VOCAB = 1000000
EMB_DIM = 64
BATCH = 1024
SEQ = 200

NUM_IDX = BATCH * SEQ
WINDOW = 128


def _gather_fn():
    mesh = plsc.VectorSubcoreMesh(
        core_axis_name="core", subcore_axis_name="subcore"
    )

    @jax.jit
    def gather(table, indices):
        indices = indices.reshape((1, NUM_IDX))

        @pl.kernel(
            out_type=jax.ShapeDtypeStruct((NUM_IDX, EMB_DIM), table.dtype),
            mesh=mesh,
            compiler_params=pltpu.CompilerParams(use_tc_tiling_on_sc=False),
        )
        def kernel(x_hbm, i_hbm, o_hbm):
            def body(i_vmem, o_vmem):
                pltpu.sync_copy(x_hbm.at[i_vmem.at[0]], o_vmem)

            pltpu.emit_pipeline(
                body,
                grid=(NUM_IDX // WINDOW,),
                in_specs=[
                    pl.BlockSpec((1, WINDOW), index_map=lambda i: (0, i))
                ],
                out_specs=[
                    pl.BlockSpec((WINDOW, EMB_DIM), index_map=lambda i: (i, 0))
                ],
                core_axis_name=("core", "subcore"),
                dimension_semantics=(pltpu.PARALLEL,),
            )(i_hbm, o_hbm)

        return kernel(table, indices)

    return gather


_gather = _gather_fn()


def kernel(inp, emb_weight):
    flat_idx = inp.reshape(-1)
    out = _gather(emb_weight, flat_idx)
    return out.reshape(BATCH, SEQ, EMB_DIM)


# speedup vs baseline: 1.0318x; 1.0318x over previous
import jax
import jax.numpy as jnp
from jax import lax
from jax.experimental import pallas as pl
from jax.experimental.pallas import tpu as pltpu
from jax.experimental.pallas import tpu_sc as plsc

VOCAB = 1000000
EMB_DIM = 64
BATCH = 1024
SEQ = 200

NUM_IDX = BATCH * SEQ
WINDOW = 128
NUM_CHUNKS = NUM_IDX // WINDOW
NW = 32
CHUNKS_PER_W = NUM_CHUNKS // NW
NBUF = 5


def _gather_fn():
    mesh = plsc.VectorSubcoreMesh(
        core_axis_name="core", subcore_axis_name="subcore"
    )

    @jax.jit
    def gather(table, indices):
        idx2d = indices.reshape((NUM_CHUNKS, WINDOW))

        @pl.kernel(
            out_type=jax.ShapeDtypeStruct((NUM_IDX, EMB_DIM), table.dtype),
            mesh=mesh,
            scratch_types=[
                pltpu.VMEM((CHUNKS_PER_W, WINDOW), jnp.int32),
                pltpu.VMEM((NBUF, WINDOW, EMB_DIM), jnp.float32),
                pltpu.SemaphoreType.DMA,
                pltpu.SemaphoreType.DMA((NBUF,)),
                pltpu.SemaphoreType.DMA((NBUF,)),
            ],
            compiler_params=pltpu.CompilerParams(use_tc_tiling_on_sc=False),
        )
        def kernel(x_hbm, i_hbm, o_hbm, idx_v, rows_v, isem, gsem, ssem):
            cid = lax.axis_index("core")
            sid = lax.axis_index("subcore")
            wid = sid * 2 + cid
            row0 = wid * CHUNKS_PER_W

            pltpu.async_copy(
                i_hbm.at[pl.ds(row0, CHUNKS_PER_W)], idx_v, isem
            ).wait()

            for b in range(NBUF):
                pltpu.async_copy(x_hbm.at[idx_v.at[b]], rows_v.at[b], gsem.at[b])

            @pl.loop(0, CHUNKS_PER_W, step=NBUF)
            def _(j0):
                for b in range(NBUF):
                    j = j0 + b
                    pltpu.make_async_copy(
                        x_hbm.at[idx_v.at[b]], rows_v.at[b], gsem.at[b]
                    ).wait()
                    out_slice = o_hbm.at[pl.ds((row0 + j) * WINDOW, WINDOW)]
                    pltpu.async_copy(rows_v.at[b], out_slice, ssem.at[b])
                    pltpu.make_async_copy(
                        rows_v.at[b], out_slice, ssem.at[b]
                    ).wait()

                    @pl.when(j + NBUF < CHUNKS_PER_W)
                    def _():
                        pltpu.async_copy(
                            x_hbm.at[idx_v.at[j + NBUF]],
                            rows_v.at[b],
                            gsem.at[b],
                        )

        return kernel(table, idx2d)

    return gather


_gather = _gather_fn()


def kernel(inp, emb_weight):
    flat_idx = inp.reshape(-1)
    out = _gather(emb_weight, flat_idx)
    return out.reshape(BATCH, SEQ, EMB_DIM)
